# baseline (speedup 1.0000x reference)
import jax
import jax.numpy as jnp
from jax import lax
from jax.experimental import pallas as pl
from jax.experimental.pallas import tpu as pltpu
from jax.experimental.pallas import tpu_sc as plsc

B = 1024
T = 32768
N = 4096
NB = 16
WPN = 2115
NC, NS = 2, 16
NW = NC * NS
NPW = N // NW
KCH = 16
GRP = 8
GW = GRP * WPN
BW = B // 4


def _pack_body(x_ref, out_ref):
    bits = (x_ref[...] > 0.5).astype(jnp.int32)
    r = bits.reshape(B // 4, 4, x_ref.shape[1])
    packed = (r[:, 0, :] | (r[:, 1, :] << 8) |
              (r[:, 2, :] << 16) | (r[:, 3, :] << 24))
    out_ref[...] = packed.T


def _pack_bits(x):
    TB = 512
    return pl.pallas_call(
        _pack_body,
        grid=(T // TB,),
        in_specs=[pl.BlockSpec((B, TB), lambda i: (i - i, i))],
        out_specs=pl.BlockSpec((TB, BW), lambda i: (i, i - i)),
        out_shape=jax.ShapeDtypeStruct((T, BW), jnp.int32),
    )(x)


_SC_PARAMS = pltpu.CompilerParams(use_tc_tiling_on_sc=False,
                                  needs_layout_passes=False)


def _mem_body(bits_hbm, conn_hbm, mlo_hbm, mhi_hbm, out_hbm,
              conn_v, rows0_v, rows1_v, tbl0_v, tbl1_v, abuf_v, buf_v,
              sr0, sr1, st0, st1):
    i32 = jnp.int32
    cid = lax.axis_index("c").astype(i32)
    sid = lax.axis_index("s").astype(i32)
    wid = sid * i32(NC) + cid
    n0 = wid * i32(NPW)
    g0 = n0 // i32(GRP)
    pltpu.sync_copy(conn_hbm.at[pl.ds(n0, NPW)], conn_v)
    iota = lax.iota(jnp.int32, 16)
    iota4 = iota * 4

    def issue_rows(nl, rows_v, sr):
        pltpu.async_copy(bits_hbm.at[conn_v.at[nl]], rows_v, sr)

    def wait_rows(rows_v, sr):
        pltpu.make_async_copy(bits_hbm.at[conn_v.at[i32(0)]], rows_v,
                              sr).wait()

    def issue_tbl(gi, tbl_v, st):
        pltpu.async_copy(mlo_hbm.at[gi], tbl_v.at[i32(0)], st)
        pltpu.async_copy(mhi_hbm.at[gi], tbl_v.at[i32(1)], st)

    def wait_tbl(tbl_v, st):
        pltpu.make_async_copy(mlo_hbm.at[g0], tbl_v.at[i32(0)], st).wait()
        pltpu.make_async_copy(mhi_hbm.at[g0], tbl_v.at[i32(1)], st).wait()

    issue_tbl(g0, tbl0_v, st0)
    issue_tbl(g0 + i32(1), tbl1_v, st1)
    issue_rows(i32(0), rows0_v, sr0)

    def make_addr_group(rows_v, ncv):
        def group_body(g, c):
            base = g * i32(16)
            lo32 = jnp.zeros((16,), jnp.int32)
            hi32 = jnp.zeros((16,), jnp.int32)
            for j in range(8):
                wlo = rows_v[j, pl.ds(base, 16)]
                whi = rows_v[j + 8, pl.ds(base, 16)]
                lo32 = lo32 + lax.shift_left(wlo, jnp.int32(j))
                hi32 = hi32 + lax.shift_left(whi, jnp.int32(j))
            for k in range(4):
                lob = lax.shift_right_logical(lo32, jnp.int32(8 * k)) & 255
                hib = lax.shift_right_logical(hi32, jnp.int32(8 * k)) & 255
                a = lob + (hib << 8)
                plsc.store_scatter(
                    abuf_v, [ncv, iota4 + (g * i32(64) + i32(k))], a)
            return c
        return group_body

    def addr_neuron(nl, nc, rows_v, rows_nxt, sr_cur, sr_nxt):
        @pl.when(nl + i32(1) < i32(NPW))
        def _():
            issue_rows(nl + i32(1), rows_nxt, sr_nxt)
        wait_rows(rows_v, sr_cur)
        ncv = jnp.zeros((16,), i32) + i32(nc)
        lax.fori_loop(i32(0), i32(16), make_addr_group(rows_v, ncv), i32(0))

    def lookup_neuron(nc, tbl_v):
        ncv = jnp.zeros((16,), i32) + i32(nc)
        toff = i32((nc % GRP) * WPN)

        def group_body(g, c):
            a = abuf_v[nc, pl.ds(g * i32(16), 16)]
            q = lax.shift_right_logical(a * 2115, jnp.int32(16))
            r = a - q * 31
            neg = r < 0
            q = jnp.where(neg, q - 1, q)
            r = jnp.where(neg, r + 31, r)
            half = (r >= 16).astype(jnp.int32)
            w = plsc.load_gather(tbl_v, [half, toff + q])
            val = lax.shift_right_logical(w, (r & 15) * 2) & 3
            plsc.store_scatter(buf_v, [g * i32(16) + iota, ncv], val)
            return c

        lax.fori_loop(i32(0), i32(B // 16), group_body, i32(0))

    def chunk_body(ch, carry):
        base_n = ch * i32(KCH)
        for p in range(KCH // 2):
            nl = base_n + i32(2 * p)
            addr_neuron(nl, 2 * p, rows0_v, rows1_v, sr0, sr1)
            addr_neuron(nl + i32(1), 2 * p + 1, rows1_v, rows0_v, sr1, sr0)
        wait_tbl(tbl0_v, st0)
        for nc in range(GRP):
            lookup_neuron(nc, tbl0_v)
        @pl.when(ch + i32(1) < i32(NPW // KCH))
        def _():
            issue_tbl(g0 + (ch + i32(1)) * i32(2), tbl0_v, st0)
        wait_tbl(tbl1_v, st1)
        for nc in range(GRP, KCH):
            lookup_neuron(nc, tbl1_v)
        @pl.when(ch + i32(1) < i32(NPW // KCH))
        def _():
            issue_tbl(g0 + (ch + i32(1)) * i32(2) + i32(1), tbl1_v, st1)
        pltpu.sync_copy(buf_v,
                        out_hbm.at[:, pl.ds(n0 + base_n, KCH)])
        return carry

    lax.fori_loop(i32(0), i32(NPW // KCH), chunk_body, i32(0))


def _mem_call(bitsT, conn32, mlo, mhi):
    mesh = plsc.VectorSubcoreMesh(core_axis_name="c", subcore_axis_name="s")
    f = pl.kernel(
        _mem_body,
        out_type=jax.ShapeDtypeStruct((B, N), jnp.int32),
        mesh=mesh,
        scratch_types=[
            pltpu.VMEM((NPW, NB), jnp.int32),
            pltpu.VMEM((NB, BW), jnp.int32),
            pltpu.VMEM((NB, BW), jnp.int32),
            pltpu.VMEM((2, GW), jnp.int32),
            pltpu.VMEM((2, GW), jnp.int32),
            pltpu.VMEM((KCH, B), jnp.int32),
            pltpu.VMEM((B, KCH), jnp.int32),
            pltpu.SemaphoreType.DMA,
            pltpu.SemaphoreType.DMA,
            pltpu.SemaphoreType.DMA,
            pltpu.SemaphoreType.DMA,
        ],
        compiler_params=_SC_PARAMS,
    )
    return f(bitsT, conn32, mlo, mhi)


def kernel(x, connections, memory_words):
    conn32 = connections.astype(jnp.int32)
    bitsT = _pack_bits(x)
    mlo = memory_words.astype(jnp.int32).reshape(N // GRP, GW)
    mhi = lax.shift_right_logical(memory_words, jnp.int64(32)).astype(
        jnp.int32).reshape(N // GRP, GW)
    out32 = _mem_call(bitsT, conn32, mlo, mhi)
    return out32.astype(jnp.uint32).astype(jnp.int64)


# speedup vs baseline: 1.0524x; 1.0524x over previous
import jax
import jax.numpy as jnp
from jax import lax
from jax.experimental import pallas as pl
from jax.experimental.pallas import tpu as pltpu
from jax.experimental.pallas import tpu_sc as plsc

B = 1024
T = 32768
N = 4096
NB = 16
WPN = 2115
NC, NS = 2, 16
NW = NC * NS
NPW = N // NW
KCH = 16
GRP = 8
BW = B // 4


def _pack_body(x_ref, out_ref):
    bits = (x_ref[...] > 0.5).astype(jnp.int32)
    r = bits.reshape(B // 4, 4, x_ref.shape[1])
    packed = (r[:, 0, :] | (r[:, 1, :] << 8) |
              (r[:, 2, :] << 16) | (r[:, 3, :] << 24))
    out_ref[...] = packed.T


def _pack_bits(x):
    TB = 512
    return pl.pallas_call(
        _pack_body,
        grid=(T // TB,),
        in_specs=[pl.BlockSpec((B, TB), lambda i: (i - i, i))],
        out_specs=pl.BlockSpec((TB, BW), lambda i: (i, i - i)),
        out_shape=jax.ShapeDtypeStruct((T, BW), jnp.int32),
    )(x)


_SC_PARAMS = pltpu.CompilerParams(use_tc_tiling_on_sc=False,
                                  needs_layout_passes=False)


def _addr_body(bits_hbm, conn_hbm, addr_hbm, conn_v, rows0_v, rows1_v,
               buf_v, sr0, sr1):
    i32 = jnp.int32
    cid = lax.axis_index("c").astype(i32)
    sid = lax.axis_index("s").astype(i32)
    wid = sid * i32(NC) + cid
    n0 = wid * i32(NPW)
    pltpu.sync_copy(conn_hbm.at[pl.ds(n0, NPW)], conn_v)
    iota4 = lax.iota(jnp.int32, 16) * 4

    def issue_rows(nl, rows_v, sr):
        pltpu.async_copy(bits_hbm.at[conn_v.at[nl]], rows_v, sr)

    def wait_rows(rows_v, sr):
        pltpu.make_async_copy(bits_hbm.at[conn_v.at[i32(0)]], rows_v,
                              sr).wait()

    issue_rows(i32(0), rows0_v, sr0)

    def make_group_body(rows_v, ncv):
        def group_body(g, c):
            base = g * i32(16)
            lo32 = jnp.zeros((16,), jnp.int32)
            hi32 = jnp.zeros((16,), jnp.int32)
            for j in range(8):
                wlo = rows_v[j, pl.ds(base, 16)]
                whi = rows_v[j + 8, pl.ds(base, 16)]
                lo32 = lo32 + lax.shift_left(wlo, jnp.int32(j))
                hi32 = hi32 + lax.shift_left(whi, jnp.int32(j))
            for k in range(4):
                lob = lax.shift_right_logical(lo32, jnp.int32(8 * k)) & 255
                hib = lax.shift_right_logical(hi32, jnp.int32(8 * k)) & 255
                a = lob + (hib << 8)
                plsc.store_scatter(
                    buf_v, [ncv, iota4 + (g * i32(64) + i32(k))], a)
            return c
        return group_body

    def do_neuron(nl, rows_v, rows_nxt, sr_cur, sr_nxt):
        @pl.when(nl + i32(1) < i32(NPW))
        def _():
            issue_rows(nl + i32(1), rows_nxt, sr_nxt)
        wait_rows(rows_v, sr_cur)
        ncv = jnp.zeros((16,), i32) + (nl & i32(KCH - 1))
        lax.fori_loop(i32(0), i32(16), make_group_body(rows_v, ncv), i32(0))

    def chunk_body(ch, carry):
        for p in range(KCH // 2):
            nl = ch * i32(KCH) + i32(2 * p)
            do_neuron(nl, rows0_v, rows1_v, sr0, sr1)
            do_neuron(nl + i32(1), rows1_v, rows0_v, sr1, sr0)
        pltpu.sync_copy(buf_v,
                        addr_hbm.at[pl.ds(n0 + ch * i32(KCH), KCH)])
        return carry

    lax.fori_loop(i32(0), i32(NPW // KCH), chunk_body, i32(0))


def _addr_call(bitsT, conn32):
    mesh = plsc.VectorSubcoreMesh(core_axis_name="c", subcore_axis_name="s")
    f = pl.kernel(
        _addr_body,
        out_type=jax.ShapeDtypeStruct((N, B), jnp.int32),
        mesh=mesh,
        scratch_types=[
            pltpu.VMEM((NPW, NB), jnp.int32),
            pltpu.VMEM((NB, BW), jnp.int32),
            pltpu.VMEM((NB, BW), jnp.int32),
            pltpu.VMEM((KCH, B), jnp.int32),
            pltpu.SemaphoreType.DMA,
            pltpu.SemaphoreType.DMA,
        ],
        compiler_params=_SC_PARAMS,
    )
    return f(bitsT, conn32)


def _lookup_body(addr_hbm, mlo_hbm, mhi_hbm, out_hbm,
                 tbl0_v, tbl1_v, arow0_v, arow1_v, buf_v,
                 st0, st1, sa0, sa1):
    i32 = jnp.int32
    cid = lax.axis_index("c").astype(i32)
    sid = lax.axis_index("s").astype(i32)
    wid = sid * i32(NC) + cid
    n0 = wid * i32(NPW)
    g0 = n0 // i32(GRP)
    iota = lax.iota(jnp.int32, 16)
    ngrp = NPW // GRP

    def issue_tbl(gi, tbl_v, st):
        pltpu.async_copy(mlo_hbm.at[gi], tbl_v.at[i32(0)], st)
        pltpu.async_copy(mhi_hbm.at[gi], tbl_v.at[i32(1)], st)

    def wait_tbl(tbl_v, st):
        pltpu.make_async_copy(mlo_hbm.at[g0], tbl_v.at[i32(0)], st).wait()
        pltpu.make_async_copy(mhi_hbm.at[g0], tbl_v.at[i32(1)], st).wait()

    def issue_arow(n, arow_v, sa):
        pltpu.async_copy(addr_hbm.at[n0 + n], arow_v, sa)

    def wait_arow(arow_v, sa):
        pltpu.make_async_copy(addr_hbm.at[n0], arow_v, sa).wait()

    issue_tbl(g0, tbl0_v, st0)
    issue_arow(i32(0), arow0_v, sa0)

    def make_group_body(tbl_v, arow_v, ncv, toff):
        def group_body(g, c):
            a = arow_v[pl.ds(g * i32(16), 16)]
            q = lax.shift_right_logical(a * 2115, jnp.int32(16))
            r = a - q * 31
            neg = r < 0
            q = jnp.where(neg, q - 1, q)
            r = jnp.where(neg, r + 31, r)
            half = (r >= 16).astype(jnp.int32)
            w = plsc.load_gather(tbl_v, [half, toff + q])
            val = lax.shift_right_logical(w, (r & 15) * 2) & 3
            plsc.store_scatter(buf_v, [g * i32(16) + iota, ncv], val)
            return c
        return group_body

    def do_neuron(n, nl, tbl_v, arow_v, arow_nxt, sa, sa_nxt):
        @pl.when(n + i32(1) < i32(NPW))
        def _():
            issue_arow(n + i32(1), arow_nxt, sa_nxt)
        wait_arow(arow_v, sa)
        ncv = jnp.zeros((16,), i32) + (n & i32(KCH - 1))
        lax.fori_loop(i32(0), i32(B // 16),
                      make_group_body(tbl_v, arow_v, ncv, nl * i32(WPN)),
                      i32(0))

    def do_grp(grp, tbl_v, arow_slots):
        for nl in range(GRP):
            n = grp * i32(GRP) + i32(nl)
            a_cur, a_nxt, s_cur, s_nxt = arow_slots[nl % 2]
            do_neuron(n, nl, tbl_v, a_cur, a_nxt, s_cur, s_nxt)

    slots_even = {0: (arow0_v, arow1_v, sa0, sa1),
                  1: (arow1_v, arow0_v, sa1, sa0)}

    def chunk_body(ch, carry):
        grp_a = ch * i32(2)
        grp_b = grp_a + i32(1)
        issue_tbl(g0 + grp_b, tbl1_v, st1)
        wait_tbl(tbl0_v, st0)
        do_grp(grp_a, tbl0_v, slots_even)
        @pl.when(grp_b + i32(1) < i32(ngrp))
        def _():
            issue_tbl(g0 + grp_b + i32(1), tbl0_v, st0)
        wait_tbl(tbl1_v, st1)
        do_grp(grp_b, tbl1_v, slots_even)
        pltpu.sync_copy(buf_v,
                        out_hbm.at[:, pl.ds(n0 + ch * i32(KCH), KCH)])
        return carry

    lax.fori_loop(i32(0), i32(NPW // KCH), chunk_body, i32(0))


def _lookup_call(addr, mlo, mhi):
    mesh = plsc.VectorSubcoreMesh(core_axis_name="c", subcore_axis_name="s")
    f = pl.kernel(
        _lookup_body,
        out_type=jax.ShapeDtypeStruct((B, N), jnp.int32),
        mesh=mesh,
        scratch_types=[
            pltpu.VMEM((2, GRP * WPN), jnp.int32),
            pltpu.VMEM((2, GRP * WPN), jnp.int32),
            pltpu.VMEM((B,), jnp.int32),
            pltpu.VMEM((B,), jnp.int32),
            pltpu.VMEM((B, KCH), jnp.int32),
            pltpu.SemaphoreType.DMA,
            pltpu.SemaphoreType.DMA,
            pltpu.SemaphoreType.DMA,
            pltpu.SemaphoreType.DMA,
        ],
        compiler_params=_SC_PARAMS,
    )
    return f(addr, mlo, mhi)


def kernel(x, connections, memory_words):
    conn32 = connections.astype(jnp.int32)
    bitsT = _pack_bits(x)
    addr = _addr_call(bitsT, conn32)
    mlo = memory_words.astype(jnp.int32).reshape(N // GRP, GRP * WPN)
    mhi = lax.shift_right_logical(memory_words, jnp.int64(32)).astype(
        jnp.int32).reshape(N // GRP, GRP * WPN)
    out32 = _lookup_call(addr, mlo, mhi)
    return out32.astype(jnp.uint32).astype(jnp.int64)
